# baseline (speedup 1.0000x reference)
import functools

import jax
import jax.numpy as jnp
from jax import lax
from jax.experimental import pallas as pl
from jax.experimental.pallas import tpu as pltpu
from jax.experimental.pallas import tpu_sc as plsc

N = 10000
E = 320000
DIN = 128
DH = 64
G = 64

NC = 2
NS = 16
NW = NC * NS
LANE = 128
C = 80
NBUF = 4
EPAD = NW * C * LANE
NPAD = 10112
RPS = NPAD // NS

_mesh = plsc.VectorSubcoreMesh(core_axis_name="c", subcore_axis_name="s")


@functools.partial(
    pl.kernel, mesh=_mesh,
    compiler_params=pltpu.CompilerParams(use_tc_tiling_on_sc=False),
    out_type=jax.ShapeDtypeStruct((NC, NPAD, 16), jnp.float32),
    scratch_types=[
        pltpu.VMEM((C, LANE), jnp.int32),
        pltpu.VMEM((LANE, 16), jnp.float32),
        pltpu.VMEM_SHARED((NPAD, 16), jnp.float32),
    ],
)
def _sc_degree(dstb_hbm, ones_hbm, zeros_hbm, out_hbm, idx_d, ones_v, acc):
    c = lax.axis_index("c")
    s = lax.axis_index("s")
    wid = s * NC + c
    pltpu.sync_copy(zeros_hbm.at[pl.ds(s * RPS, RPS)], acc.at[pl.ds(s * RPS, RPS)])
    pltpu.sync_copy(dstb_hbm.at[wid], idx_d)
    pltpu.sync_copy(ones_hbm, ones_v)
    plsc.subcore_barrier()

    def body(i, carry):
        pltpu.sync_copy(ones_v, acc.at[idx_d.at[i]], add=True)
        return carry

    lax.fori_loop(0, C, body, 0)
    plsc.subcore_barrier()
    pltpu.sync_copy(acc.at[pl.ds(s * RPS, RPS)], out_hbm.at[c, pl.ds(s * RPS, RPS)])


@functools.partial(
    pl.kernel, mesh=_mesh,
    compiler_params=pltpu.CompilerParams(use_tc_tiling_on_sc=False),
    out_type=jax.ShapeDtypeStruct((NC, NPAD, DH), jnp.float32),
    scratch_types=[
        pltpu.VMEM((C, LANE), jnp.int32),
        pltpu.VMEM((C, LANE), jnp.int32),
        pltpu.VMEM_SHARED((NPAD, DH), jnp.float32),
    ] + [pltpu.VMEM((LANE, DH), jnp.float32) for _ in range(NBUF)]
      + [pltpu.SemaphoreType.DMA for _ in range(NBUF)],
)
def _sc_scatter(g_hbm, srcb_hbm, dstb_hbm, zeros_hbm, out_hbm,
                idx_s, idx_d, acc, *rows_and_sems):
    rows = rows_and_sems[:NBUF]
    sems = rows_and_sems[NBUF:]
    c = lax.axis_index("c")
    s = lax.axis_index("s")
    wid = s * NC + c
    pltpu.sync_copy(zeros_hbm.at[pl.ds(s * RPS, RPS)], acc.at[pl.ds(s * RPS, RPS)])
    pltpu.sync_copy(srcb_hbm.at[wid], idx_s)
    pltpu.sync_copy(dstb_hbm.at[wid], idx_d)
    plsc.subcore_barrier()

    for b in range(NBUF):
        pltpu.async_copy(g_hbm.at[idx_s.at[b]], rows[b], sems[b])

    def body(gi, carry):
        for b in range(NBUF):
            i = gi * NBUF + b
            pltpu.make_async_copy(g_hbm.at[idx_s.at[i]], rows[b], sems[b]).wait()
            pltpu.sync_copy(rows[b], acc.at[idx_d.at[i]], add=True)
            nxt = i + NBUF

            @pl.when(nxt < C)
            def _():
                pltpu.async_copy(g_hbm.at[idx_s.at[nxt]], rows[b], sems[b])
        return carry

    lax.fori_loop(0, C // NBUF, body, 0)
    plsc.subcore_barrier()
    pltpu.sync_copy(acc.at[pl.ds(s * RPS, RPS)], out_hbm.at[c, pl.ds(s * RPS, RPS)])


def _dinv_body(degp_ref, dinv_ref):
    deg = degp_ref[0, :, 0:1] + degp_ref[1, :, 0:1] + 1.0
    row = lax.broadcasted_iota(jnp.int32, (NPAD, 1), 0)
    dinv_ref[...] = jnp.where(row < N, lax.rsqrt(deg), 0.0)


_tc_dinv = pl.pallas_call(
    _dinv_body, out_shape=jax.ShapeDtypeStruct((NPAD, 1), jnp.float32))


def _l0_body(x_ref, w_ref, dinv_ref, g_ref):
    h = jnp.dot(x_ref[...], w_ref[...], preferred_element_type=jnp.float32)
    g_ref[...] = dinv_ref[...] * h


_tc_l0 = pl.pallas_call(
    _l0_body, out_shape=jax.ShapeDtypeStruct((NPAD, DH), jnp.float32))


def _mid_body(s_ref, gp_ref, dinv_ref, b_ref, w_ref, g_ref):
    t = s_ref[0] + s_ref[1] + gp_ref[...]
    a = jnp.maximum(dinv_ref[...] * t + b_ref[...], 0.0)
    h = jnp.dot(a, w_ref[...], preferred_element_type=jnp.float32)
    g_ref[...] = dinv_ref[...] * h


_tc_mid = pl.pallas_call(
    _mid_body, out_shape=jax.ShapeDtypeStruct((NPAD, DH), jnp.float32))


def _final_body(s_ref, gp_ref, dinv_ref, b_ref, batch_ref, wl_ref, bl_ref, out_ref):
    t = s_ref[0] + s_ref[1] + gp_ref[...]
    a = jnp.maximum(dinv_ref[...] * t + b_ref[...], 0.0)
    gid = lax.broadcasted_iota(jnp.int32, (NPAD, G), 1)
    onehot = (batch_ref[...] == gid).astype(jnp.float32)
    sums = lax.dot_general(onehot, a, (((0,), (0,)), ((), ())),
                           preferred_element_type=jnp.float32)
    ones_col = jnp.ones((NPAD, 1), jnp.float32)
    cnt = lax.dot_general(onehot, ones_col, (((0,), (0,)), ((), ())),
                          preferred_element_type=jnp.float32)
    pooled = sums / jnp.maximum(cnt, 1.0)
    out_ref[...] = jnp.dot(pooled, wl_ref[...],
                           preferred_element_type=jnp.float32) + bl_ref[...]


_tc_final = pl.pallas_call(
    _final_body, out_shape=jax.ShapeDtypeStruct((G, DOUT := 128), jnp.float32))


def kernel(x, edge_index, batch, W1, b1, Wh, bh, Wlin, blin):
    src = edge_index[0]
    dst = edge_index[1]
    pad = jnp.full((EPAD - E,), N, dtype=jnp.int32)
    srcb = jnp.concatenate([src, pad]).reshape(NW, C, LANE)
    dstb = jnp.concatenate([dst, pad]).reshape(NW, C, LANE)
    x_p = jnp.pad(x, ((0, NPAD - N), (0, 0)))
    batch_p = jnp.pad(batch, (0, NPAD - N), constant_values=G).reshape(NPAD, 1)
    zeros64 = jnp.zeros((NPAD, DH), jnp.float32)
    zeros16 = jnp.zeros((NPAD, 16), jnp.float32)
    ones16 = jnp.ones((LANE, 16), jnp.float32)

    degp = _sc_degree(dstb, ones16, zeros16)
    dinv = _tc_dinv(degp)
    g = _tc_l0(x_p, W1, dinv)
    biases = [b1] + [bh[i] for i in range(8)]
    for i in range(9):
        s = _sc_scatter(g, srcb, dstb, zeros64)
        g = _tc_mid(s, g, dinv, biases[i].reshape(1, DH), Wh[i])
    s = _sc_scatter(g, srcb, dstb, zeros64)
    out = _tc_final(s, g, dinv, bh[8].reshape(1, DH), batch_p,
                    Wlin, blin.reshape(1, 128))
    return out


# speedup vs baseline: 2.0212x; 2.0212x over previous
import functools

import jax
import jax.numpy as jnp
from jax import lax
from jax.experimental import pallas as pl
from jax.experimental.pallas import tpu as pltpu
from jax.experimental.pallas import tpu_sc as plsc

N = 10000
E = 320000
DIN = 128
DH = 64
G = 64

NC = 2
NS = 16
NW = NC * NS
LANE = 128
C = 80
NBUF = 2
EPAD = NW * C * LANE
NPAD = 10112
RPS = NPAD // NS

_mesh = plsc.VectorSubcoreMesh(core_axis_name="c", subcore_axis_name="s")


@functools.partial(
    pl.kernel, mesh=_mesh,
    compiler_params=pltpu.CompilerParams(use_tc_tiling_on_sc=False),
    out_type=jax.ShapeDtypeStruct((NC, NPAD, 16), jnp.float32),
    scratch_types=[
        pltpu.VMEM((C, LANE), jnp.int32),
        pltpu.VMEM((LANE, 16), jnp.float32),
        pltpu.VMEM_SHARED((NPAD, 16), jnp.float32),
    ],
)
def _sc_degree(dstb_hbm, ones_hbm, zeros_hbm, out_hbm, idx_d, ones_v, acc):
    c = lax.axis_index("c")
    s = lax.axis_index("s")
    wid = s * NC + c
    pltpu.sync_copy(zeros_hbm.at[pl.ds(s * RPS, RPS)], acc.at[pl.ds(s * RPS, RPS)])
    pltpu.sync_copy(dstb_hbm.at[wid], idx_d)
    pltpu.sync_copy(ones_hbm, ones_v)
    plsc.subcore_barrier()

    def body(i, carry):
        pltpu.sync_copy(ones_v, acc.at[idx_d.at[i]], add=True)
        return carry

    lax.fori_loop(0, C, body, 0)
    plsc.subcore_barrier()
    pltpu.sync_copy(acc.at[pl.ds(s * RPS, RPS)], out_hbm.at[c, pl.ds(s * RPS, RPS)])


@functools.partial(
    pl.kernel, mesh=_mesh,
    compiler_params=pltpu.CompilerParams(use_tc_tiling_on_sc=False),
    out_type=jax.ShapeDtypeStruct((NC, NPAD, DH), jnp.float32),
    scratch_types=[
        pltpu.VMEM((C, LANE), jnp.int32),
        pltpu.VMEM((C, LANE), jnp.int32),
        pltpu.VMEM_SHARED((NPAD, DH), jnp.float32),
        pltpu.VMEM_SHARED((NPAD, DH), jnp.float32),
    ] + [pltpu.VMEM((LANE, DH), jnp.float32) for _ in range(NBUF)]
      + [pltpu.SemaphoreType.DMA for _ in range(NBUF)],
)
def _sc_scatter(g_hbm, srcb_hbm, dstb_hbm, zeros_hbm, out_hbm,
                idx_s, idx_d, acc, g_sp, *rows_and_sems):
    rows = rows_and_sems[:NBUF]
    sems = rows_and_sems[NBUF:]
    c = lax.axis_index("c")
    s = lax.axis_index("s")
    wid = s * NC + c

    def zbody(k, carry):
        r0 = (s + NS * k) * LANE

        @pl.when(r0 < NPAD)
        def _():
            pltpu.sync_copy(zeros_hbm, acc.at[pl.ds(r0, LANE)])
        return carry

    lax.fori_loop(0, (NPAD // LANE + NS - 1) // NS, zbody, 0)
    pltpu.sync_copy(g_hbm.at[pl.ds(s * RPS, RPS)], g_sp.at[pl.ds(s * RPS, RPS)])
    pltpu.sync_copy(srcb_hbm.at[wid], idx_s)
    pltpu.sync_copy(dstb_hbm.at[wid], idx_d)
    plsc.subcore_barrier()

    for b in range(NBUF):
        pltpu.async_copy(g_sp.at[idx_s.at[b]], rows[b], sems[b])

    def body(gi, carry):
        for b in range(NBUF):
            i = gi * NBUF + b
            pltpu.make_async_copy(g_sp.at[idx_s.at[i]], rows[b], sems[b]).wait()
            pltpu.sync_copy(rows[b], acc.at[idx_d.at[i]], add=True)
            nxt = i + NBUF

            @pl.when(nxt < C)
            def _():
                pltpu.async_copy(g_sp.at[idx_s.at[nxt]], rows[b], sems[b])
        return carry

    lax.fori_loop(0, C // NBUF, body, 0)
    plsc.subcore_barrier()
    pltpu.sync_copy(acc.at[pl.ds(s * RPS, RPS)], out_hbm.at[c, pl.ds(s * RPS, RPS)])


def _dinv_body(degp_ref, dinv_ref):
    deg = degp_ref[0, :, 0:1] + degp_ref[1, :, 0:1] + 1.0
    row = lax.broadcasted_iota(jnp.int32, (NPAD, 1), 0)
    dinv_ref[...] = jnp.where(row < N, lax.rsqrt(deg), 0.0)


_tc_dinv = pl.pallas_call(
    _dinv_body, out_shape=jax.ShapeDtypeStruct((NPAD, 1), jnp.float32))


def _l0_body(x_ref, w_ref, dinv_ref, g_ref):
    h = jnp.dot(x_ref[...], w_ref[...], preferred_element_type=jnp.float32)
    g_ref[...] = dinv_ref[...] * h


_tc_l0 = pl.pallas_call(
    _l0_body, out_shape=jax.ShapeDtypeStruct((NPAD, DH), jnp.float32))


def _mid_body(s_ref, gp_ref, dinv_ref, b_ref, w_ref, g_ref):
    t = s_ref[0] + s_ref[1] + gp_ref[...]
    a = jnp.maximum(dinv_ref[...] * t + b_ref[...], 0.0)
    h = jnp.dot(a, w_ref[...], preferred_element_type=jnp.float32)
    g_ref[...] = dinv_ref[...] * h


_tc_mid = pl.pallas_call(
    _mid_body, out_shape=jax.ShapeDtypeStruct((NPAD, DH), jnp.float32))


def _final_body(s_ref, gp_ref, dinv_ref, b_ref, batch_ref, wl_ref, bl_ref, out_ref):
    t = s_ref[0] + s_ref[1] + gp_ref[...]
    a = jnp.maximum(dinv_ref[...] * t + b_ref[...], 0.0)
    gid = lax.broadcasted_iota(jnp.int32, (NPAD, G), 1)
    onehot = (batch_ref[...] == gid).astype(jnp.float32)
    sums = lax.dot_general(onehot, a, (((0,), (0,)), ((), ())),
                           preferred_element_type=jnp.float32)
    ones_col = jnp.ones((NPAD, 1), jnp.float32)
    cnt = lax.dot_general(onehot, ones_col, (((0,), (0,)), ((), ())),
                          preferred_element_type=jnp.float32)
    pooled = sums / jnp.maximum(cnt, 1.0)
    out_ref[...] = jnp.dot(pooled, wl_ref[...],
                           preferred_element_type=jnp.float32) + bl_ref[...]


_tc_final = pl.pallas_call(
    _final_body, out_shape=jax.ShapeDtypeStruct((G, DOUT := 128), jnp.float32))


def kernel(x, edge_index, batch, W1, b1, Wh, bh, Wlin, blin):
    src = edge_index[0]
    dst = edge_index[1]
    pad = jnp.full((EPAD - E,), N, dtype=jnp.int32)
    srcb = jnp.concatenate([src, pad]).reshape(NW, C, LANE)
    dstb = jnp.concatenate([dst, pad]).reshape(NW, C, LANE)
    x_p = jnp.pad(x, ((0, NPAD - N), (0, 0)))
    batch_p = jnp.pad(batch, (0, NPAD - N), constant_values=G).reshape(NPAD, 1)
    zeros64 = jnp.zeros((LANE, DH), jnp.float32)
    zeros16 = jnp.zeros((NPAD, 16), jnp.float32)
    ones16 = jnp.ones((LANE, 16), jnp.float32)

    degp = _sc_degree(dstb, ones16, zeros16)
    dinv = _tc_dinv(degp)
    g = _tc_l0(x_p, W1, dinv)
    biases = [b1] + [bh[i] for i in range(8)]
    for i in range(9):
        s = _sc_scatter(g, srcb, dstb, zeros64)
        g = _tc_mid(s, g, dinv, biases[i].reshape(1, DH), Wh[i])
    s = _sc_scatter(g, srcb, dstb, zeros64)
    out = _tc_final(s, g, dinv, bh[8].reshape(1, DH), batch_p,
                    Wlin, blin.reshape(1, 128))
    return out


# speedup vs baseline: 2.0927x; 1.0354x over previous
import functools

import jax
import jax.numpy as jnp
from jax import lax
from jax.experimental import pallas as pl
from jax.experimental.pallas import tpu as pltpu
from jax.experimental.pallas import tpu_sc as plsc

N = 10000
E = 320000
DIN = 128
DH = 64
G = 64

NC = 2
NS = 16
NW = NC * NS
LANE = 128
C = 80
NBUF = 2
EPAD = NW * C * LANE
NPAD = 10112
RPS = NPAD // NS

_mesh = plsc.VectorSubcoreMesh(core_axis_name="c", subcore_axis_name="s")


@functools.partial(
    pl.kernel, mesh=_mesh,
    compiler_params=pltpu.CompilerParams(use_tc_tiling_on_sc=False),
    out_type=jax.ShapeDtypeStruct((NC, NPAD, 16), jnp.float32),
    scratch_types=[
        pltpu.VMEM((C, LANE), jnp.int32),
        pltpu.VMEM((LANE, 16), jnp.float32),
        pltpu.VMEM_SHARED((NPAD, 16), jnp.float32),
    ],
)
def _sc_degree(dstb_hbm, ones_hbm, zeros_hbm, out_hbm, idx_d, ones_v, acc):
    c = lax.axis_index("c")
    s = lax.axis_index("s")
    wid = s * NC + c
    pltpu.sync_copy(zeros_hbm.at[pl.ds(s * RPS, RPS)], acc.at[pl.ds(s * RPS, RPS)])
    pltpu.sync_copy(dstb_hbm.at[wid], idx_d)
    pltpu.sync_copy(ones_hbm, ones_v)
    plsc.subcore_barrier()

    def body(i, carry):
        pltpu.sync_copy(ones_v, acc.at[idx_d.at[i]], add=True)
        return carry

    lax.fori_loop(0, C, body, 0)
    plsc.subcore_barrier()
    pltpu.sync_copy(acc.at[pl.ds(s * RPS, RPS)], out_hbm.at[c, pl.ds(s * RPS, RPS)])


@functools.partial(
    pl.kernel, mesh=_mesh,
    compiler_params=pltpu.CompilerParams(use_tc_tiling_on_sc=False),
    out_type=jax.ShapeDtypeStruct((NC, NPAD, DH), jnp.float32),
    scratch_types=[
        pltpu.VMEM((C, LANE), jnp.int32),
        pltpu.VMEM((C, LANE), jnp.int32),
        pltpu.VMEM_SHARED((NPAD, DH), jnp.float32),
        pltpu.VMEM_SHARED((NPAD, DH), jnp.float32),
    ] + [pltpu.VMEM((LANE, DH), jnp.float32) for _ in range(NBUF)]
      + [pltpu.SemaphoreType.DMA for _ in range(NBUF + 1)],
)
def _sc_scatter(g_hbm, srcb_hbm, dstb_hbm, zeros_hbm, out_hbm,
                idx_s, idx_d, acc, g_sp, *rows_and_sems):
    rows = rows_and_sems[:NBUF]
    sems = rows_and_sems[NBUF:NBUF + NBUF]
    psem = rows_and_sems[2 * NBUF]
    c = lax.axis_index("c")
    s = lax.axis_index("s")
    wid = s * NC + c

    cp_g = pltpu.async_copy(g_hbm.at[pl.ds(s * RPS, RPS)],
                            g_sp.at[pl.ds(s * RPS, RPS)], psem)
    cp_s = pltpu.async_copy(srcb_hbm.at[wid], idx_s, psem)
    cp_d = pltpu.async_copy(dstb_hbm.at[wid], idx_d, psem)

    def zbody(k, carry):
        r0 = (s + NS * k) * LANE

        @pl.when(r0 < NPAD)
        def _():
            pltpu.sync_copy(zeros_hbm, acc.at[pl.ds(r0, LANE)])
        return carry

    lax.fori_loop(0, (NPAD // LANE + NS - 1) // NS, zbody, 0)
    cp_g.wait()
    cp_s.wait()
    cp_d.wait()
    plsc.subcore_barrier()

    for b in range(NBUF):
        pltpu.async_copy(g_sp.at[idx_s.at[b]], rows[b], sems[b])

    def body(gi, carry):
        for b in range(NBUF):
            i = gi * NBUF + b
            pltpu.make_async_copy(g_sp.at[idx_s.at[i]], rows[b], sems[b]).wait()
            pltpu.sync_copy(rows[b], acc.at[idx_d.at[i]], add=True)
            nxt = i + NBUF

            @pl.when(nxt < C)
            def _():
                pltpu.async_copy(g_sp.at[idx_s.at[nxt]], rows[b], sems[b])
        return carry

    lax.fori_loop(0, C // NBUF, body, 0)
    plsc.subcore_barrier()
    pltpu.sync_copy(acc.at[pl.ds(s * RPS, RPS)], out_hbm.at[c, pl.ds(s * RPS, RPS)])


def _h0_body(x_ref, w_ref, h_ref):
    h_ref[...] = jnp.dot(x_ref[...], w_ref[...],
                         preferred_element_type=jnp.float32)


_tc_h0 = pl.pallas_call(
    _h0_body, out_shape=jax.ShapeDtypeStruct((NPAD, DH), jnp.float32))


def _dinv_body(degp_ref, h_ref, dinv_ref, g_ref):
    deg = degp_ref[0, :, 0:1] + degp_ref[1, :, 0:1] + 1.0
    row = lax.broadcasted_iota(jnp.int32, (NPAD, 1), 0)
    dinv = jnp.where(row < N, lax.rsqrt(deg), 0.0)
    dinv_ref[...] = dinv
    g_ref[...] = dinv * h_ref[...]


_tc_dinv = pl.pallas_call(
    _dinv_body, out_shape=(jax.ShapeDtypeStruct((NPAD, 1), jnp.float32),
                           jax.ShapeDtypeStruct((NPAD, DH), jnp.float32)))


def _mid_body(s_ref, gp_ref, dinv_ref, b_ref, w_ref, g_ref):
    t = s_ref[0] + s_ref[1] + gp_ref[...]
    a = jnp.maximum(dinv_ref[...] * t + b_ref[...], 0.0)
    h = jnp.dot(a, w_ref[...], preferred_element_type=jnp.float32)
    g_ref[...] = dinv_ref[...] * h


_tc_mid = pl.pallas_call(
    _mid_body, out_shape=jax.ShapeDtypeStruct((NPAD, DH), jnp.float32))


def _final_body(s_ref, gp_ref, dinv_ref, b_ref, batch_ref, wl_ref, bl_ref, out_ref):
    t = s_ref[0] + s_ref[1] + gp_ref[...]
    a = jnp.maximum(dinv_ref[...] * t + b_ref[...], 0.0)
    gid = lax.broadcasted_iota(jnp.int32, (NPAD, G), 1)
    onehot = (batch_ref[...] == gid).astype(jnp.float32)
    sums = lax.dot_general(onehot, a, (((0,), (0,)), ((), ())),
                           preferred_element_type=jnp.float32)
    ones_col = jnp.ones((NPAD, 1), jnp.float32)
    cnt = lax.dot_general(onehot, ones_col, (((0,), (0,)), ((), ())),
                          preferred_element_type=jnp.float32)
    pooled = sums / jnp.maximum(cnt, 1.0)
    out_ref[...] = jnp.dot(pooled, wl_ref[...],
                           preferred_element_type=jnp.float32) + bl_ref[...]


_tc_final = pl.pallas_call(
    _final_body, out_shape=jax.ShapeDtypeStruct((G, DOUT := 128), jnp.float32))


def kernel(x, edge_index, batch, W1, b1, Wh, bh, Wlin, blin):
    src = edge_index[0]
    dst = edge_index[1]
    pad = jnp.full((EPAD - E,), N, dtype=jnp.int32)
    srcb = jnp.concatenate([src, pad]).reshape(NW, C, LANE)
    dstb = jnp.concatenate([dst, pad]).reshape(NW, C, LANE)
    x_p = jnp.pad(x, ((0, NPAD - N), (0, 0)))
    batch_p = jnp.pad(batch, (0, NPAD - N), constant_values=G).reshape(NPAD, 1)
    zeros64 = jnp.zeros((LANE, DH), jnp.float32)
    zeros16 = jnp.zeros((NPAD, 16), jnp.float32)
    ones16 = jnp.ones((LANE, 16), jnp.float32)

    degp = _sc_degree(dstb, ones16, zeros16)
    h0 = _tc_h0(x_p, W1)
    dinv, g = _tc_dinv(degp, h0)
    biases = [b1] + [bh[i] for i in range(8)]
    for i in range(9):
        s = _sc_scatter(g, srcb, dstb, zeros64)
        g = _tc_mid(s, g, dinv, biases[i].reshape(1, DH), Wh[i])
    s = _sc_scatter(g, srcb, dstb, zeros64)
    out = _tc_final(s, g, dinv, bh[8].reshape(1, DH), batch_p,
                    Wlin, blin.reshape(1, 128))
    return out
